# baseline (speedup 1.0000x reference)
import functools

import jax
import jax.numpy as jnp
from jax.experimental import pallas as pl


def _chamfer_body(a_ref, bt_ref, b2_ref, at_ref, d1_ref, d2_ref, *, num_mb):
    mb = pl.program_id(1)
    a = a_ref[0]
    bt = bt_ref[0]
    b2 = b2_ref[0]
    at = at_ref[0]

    xyn = jax.lax.dot_general(
        a[:, 0:3], bt[0:3, :], (((1,), (0,)), ((), ())),
        preferred_element_type=jnp.float32,
    )
    e = xyn + a[:, 3:4]
    d2_ref[0, 0] = jnp.min(e, axis=0) + bt[3, :]

    xynt = jax.lax.dot_general(
        b2[:, 0:3], at[0:3, :], (((1,), (0,)), ((), ())),
        preferred_element_type=jnp.float32,
    )
    ft = xynt + b2[:, 3:4]
    part1 = jnp.min(ft, axis=0)

    @pl.when(mb == 0)
    def _():
        d1_ref[0, 0] = part1

    @pl.when((mb > 0) & (mb < num_mb - 1))
    def _():
        d1_ref[0, 0] = jnp.minimum(d1_ref[0, 0], part1)

    @pl.when(mb == num_mb - 1)
    def _():
        d1_ref[0, 0] = jnp.minimum(d1_ref[0, 0], part1) + at[3, :]


def _chamfer_dists(xyz1, xyz2, *, tm=512, interpret=False):
    B, N, _ = xyz1.shape
    M = xyz2.shape[1]
    num_mb = M // tm
    x1sq = jnp.sum(xyz1 * xyz1, axis=2, keepdims=True)
    x2sq = jnp.sum(xyz2 * xyz2, axis=2, keepdims=True)
    a = jnp.concatenate([-2.0 * xyz1, x1sq], axis=2)
    b2 = jnp.concatenate([-2.0 * xyz2, x2sq], axis=2)
    bt = jnp.concatenate(
        [jnp.transpose(xyz2, (0, 2, 1)), jnp.transpose(x2sq, (0, 2, 1))],
        axis=1)
    at = jnp.concatenate(
        [jnp.transpose(xyz1, (0, 2, 1)), jnp.transpose(x1sq, (0, 2, 1))],
        axis=1)

    d1, d2 = pl.pallas_call(
        functools.partial(_chamfer_body, num_mb=num_mb),
        grid=(B, num_mb),
        in_specs=[
            pl.BlockSpec((1, N, 4), lambda b, mb: (b, 0, 0)),
            pl.BlockSpec((1, 4, tm), lambda b, mb: (b, 0, mb)),
            pl.BlockSpec((1, tm, 4), lambda b, mb: (b, mb, 0)),
            pl.BlockSpec((1, 4, N), lambda b, mb: (b, 0, 0)),
        ],
        out_specs=[
            pl.BlockSpec((1, 1, N), lambda b, mb: (b, 0, 0)),
            pl.BlockSpec((1, 1, tm), lambda b, mb: (b, 0, mb)),
        ],
        out_shape=[
            jax.ShapeDtypeStruct((B, 1, N), jnp.float32),
            jax.ShapeDtypeStruct((B, 1, M), jnp.float32),
        ],
        interpret=interpret,
    )(a, bt, b2, at)
    return d1[:, 0, :], d2[:, 0, :]


@jax.jit
def kernel(xyz1, xyz2, weights1, weights2):
    dist1, dist2 = _chamfer_dists(xyz1, xyz2)
    dist1_avg = jnp.sum(dist1 * weights1) / jnp.sum(weights1)
    dist2_avg = jnp.sum(dist2 * weights2) / jnp.sum(weights2)
    return (dist1_avg + dist2_avg) / 2.0


# speedup vs baseline: 1.3537x; 1.3537x over previous
import functools

import jax
import jax.numpy as jnp
from jax.experimental import pallas as pl
from jax.experimental.pallas import tpu as pltpu


def _chamfer_body(x1_ref, x2t_ref, d1_ref, d2_ref, acc_ref, *, num_mb, tm):
    mb = pl.program_id(1)
    a = x1_ref[0]
    bt = x2t_ref[0]
    x1sq = a[:, 3:4]
    xyn = jax.lax.dot_general(
        a[:, 0:3], bt[0:3, :], (((1,), (0,)), ((), ())),
        preferred_element_type=jnp.float32,
    )

    d2_ref[0, 0] = jnp.min(xyn + x1sq, axis=0) + bt[3, :]

    x2sq = bt[3:4, :]
    g = xyn[:, 0:128] + x2sq[:, 0:128]
    for k in range(1, tm // 128):
        sl = slice(k * 128, (k + 1) * 128)
        g = jnp.minimum(g, xyn[:, sl] + x2sq[:, sl])

    @pl.when(mb == 0)
    def _():
        acc_ref[...] = g

    @pl.when(mb > 0)
    def _():
        acc_ref[...] = jnp.minimum(acc_ref[...], g)

    @pl.when(mb == num_mb - 1)
    def _():
        d1_ref[0, 0] = jnp.min(acc_ref[...], axis=1) + x1sq[:, 0]


def _chamfer_dists(xyz1, xyz2, *, tm=512, interpret=False):
    B, N, _ = xyz1.shape
    M = xyz2.shape[1]
    num_mb = M // tm
    x1sq = jnp.sum(xyz1 * xyz1, axis=2, keepdims=True)
    a = jnp.concatenate([-2.0 * xyz1, x1sq], axis=2)
    x2t = jnp.transpose(xyz2, (0, 2, 1))
    x2sq = jnp.sum(x2t * x2t, axis=1, keepdims=True)
    bt = jnp.concatenate([x2t, x2sq], axis=1)

    d1, d2 = pl.pallas_call(
        functools.partial(_chamfer_body, num_mb=num_mb, tm=tm),
        grid=(B, num_mb),
        in_specs=[
            pl.BlockSpec((1, N, 4), lambda b, mb: (b, 0, 0)),
            pl.BlockSpec((1, 4, tm), lambda b, mb: (b, 0, mb)),
        ],
        out_specs=[
            pl.BlockSpec((1, 1, N), lambda b, mb: (b, 0, 0)),
            pl.BlockSpec((1, 1, tm), lambda b, mb: (b, 0, mb)),
        ],
        out_shape=[
            jax.ShapeDtypeStruct((B, 1, N), jnp.float32),
            jax.ShapeDtypeStruct((B, 1, M), jnp.float32),
        ],
        scratch_shapes=[pltpu.VMEM((N, 128), jnp.float32)],
        interpret=interpret,
    )(a, bt)
    return d1[:, 0, :], d2[:, 0, :]


@jax.jit
def kernel(xyz1, xyz2, weights1, weights2):
    dist1, dist2 = _chamfer_dists(xyz1, xyz2)
    dist1_avg = jnp.sum(dist1 * weights1) / jnp.sum(weights1)
    dist2_avg = jnp.sum(dist2 * weights2) / jnp.sum(weights2)
    return (dist1_avg + dist2_avg) / 2.0
